# baseline (speedup 1.0000x reference)
import functools

import numpy as np
import jax
import jax.numpy as jnp
from jax import lax
from jax.experimental import pallas as pl
from jax.experimental.pallas import tpu as pltpu
from jax.experimental.pallas import tpu_sc as plsc

_MASK_RATIO = 0.75
_B, _T, _D = 64, 1024, 768
_N_MASK = int(_T * _MASK_RATIO)
_N_KEEP = _T - _N_MASK

_ROWS = _B * _N_KEEP
_NC, _NS = 2, 16
_NW = _NC * _NS
_RPW = _ROWS // _NW
_CH = 32
_NBUF = 4
_NCHUNK = _RPW // _CH


_KEEP_IDX = np.asarray([
    2, 4, 5, 7, 16, 19, 29, 31, 34, 35, 37, 44, 45, 58, 61, 63, 65, 72,
    78, 82, 83, 85, 90, 99, 101, 102, 108, 110, 111, 112, 114, 117, 121,
    123, 129, 130, 139, 142, 144, 148, 152, 155, 156, 157, 163, 167, 174,
    175, 176, 177, 178, 179, 183, 188, 189, 197, 211, 212, 240, 251, 254,
    257, 259, 263, 268, 269, 272, 277, 278, 284, 291, 300, 302, 304, 305,
    309, 312, 315, 318, 323, 325, 336, 339, 350, 356, 363, 366, 367, 369,
    379, 388, 398, 409, 410, 415, 417, 429, 436, 441, 444, 446, 447, 448,
    452, 461, 462, 463, 480, 481, 487, 493, 495, 499, 501, 504, 507, 509,
    514, 516, 517, 518, 520, 524, 525, 532, 538, 540, 541, 542, 543, 544,
    551, 552, 553, 557, 562, 564, 565, 567, 569, 575, 577, 578, 580, 582,
    584, 585, 589, 590, 591, 598, 600, 602, 603, 605, 607, 617, 619, 638,
    649, 650, 654, 659, 670, 673, 675, 681, 690, 693, 694, 698, 703, 704,
    706, 707, 708, 709, 712, 714, 715, 730, 736, 739, 748, 750, 752, 753,
    755, 762, 765, 768, 769, 771, 774, 776, 777, 780, 787, 790, 792, 793,
    799, 803, 804, 808, 810, 816, 829, 836, 842, 846, 848, 854, 857, 859,
    864, 872, 874, 879, 883, 885, 893, 895, 901, 904, 910, 911, 914, 918,
    921, 928, 932, 934, 940, 942, 955, 957, 962, 966, 970, 973, 976, 981,
    984, 995, 996, 999, 1001, 1005, 1009, 1010, 1012, 1016, 1017, 1020,
    1021,
], dtype=np.int32)
_FLAT_IDX = (np.arange(_B, dtype=np.int64)[:, None] * _T
             + _KEEP_IDX[None, :].astype(np.int64)).reshape(-1).astype(np.int32)

_MASK_NP = np.ones(_T, dtype=bool)
_MASK_NP[_KEEP_IDX] = False
_MASK_BITS = [int(sum(int(_MASK_NP[16 * j + l]) << l for l in range(16)))
              for j in range(_T // 16)]


def _sc_gather(x_flat, flat_idx):
    mesh = plsc.VectorSubcoreMesh(core_axis_name="c", subcore_axis_name="s")

    @functools.partial(
        pl.kernel,
        out_type=(
            jax.ShapeDtypeStruct((_ROWS, _D), jnp.float32),
            jax.ShapeDtypeStruct((_T,), jnp.int32),
        ),
        mesh=mesh,
        scratch_types=[
            pltpu.VMEM((_RPW,), jnp.int32),
            pltpu.VMEM((_CH, _D), jnp.float32),
            pltpu.VMEM((_CH, _D), jnp.float32),
            pltpu.VMEM((_CH, _D), jnp.float32),
            pltpu.VMEM((_CH, _D), jnp.float32),
            pltpu.VMEM((_T,), jnp.int32),
            pltpu.SemaphoreType.DMA,
            pltpu.SemaphoreType.DMA,
            pltpu.SemaphoreType.DMA,
            pltpu.SemaphoreType.DMA,
            pltpu.SemaphoreType.DMA,
            pltpu.SemaphoreType.DMA,
            pltpu.SemaphoreType.DMA,
            pltpu.SemaphoreType.DMA,
            pltpu.SemaphoreType.DMA,
        ],
    )
    def k(x_ref, idx_ref, out_ref, mask_ref, idx_v, rows_v0, rows_v1,
          rows_v2, rows_v3, m_v, gsem0, gsem1, gsem2, gsem3, wsem0, wsem1,
          wsem2, wsem3, msem):
        wid = lax.axis_index("c") * _NS + lax.axis_index("s")
        base = wid * _RPW
        pltpu.sync_copy(idx_ref.at[pl.ds(base, _RPW)], idx_v)

        mask_copy = []

        @pl.when(wid == 0)
        def _build_mask():
            lane = lax.iota(jnp.int32, 16)
            for j in range(_T // 16):
                bits = jnp.full((16,), _MASK_BITS[j], jnp.int32)
                m_v[pl.ds(j * 16, 16)] = (bits >> lane) & 1
            mask_copy.append(pltpu.async_copy(m_v, mask_ref, msem))

        buf = [rows_v0, rows_v1, rows_v2, rows_v3]
        gsem = [gsem0, gsem1, gsem2, gsem3]
        wsem = [wsem0, wsem1, wsem2, wsem3]

        def gather(i, b):
            return pltpu.async_copy(
                x_ref.at[idx_v.at[pl.ds(i * _CH, _CH)]], buf[b], gsem[b])

        def write(i, b):
            return pltpu.async_copy(
                buf[b], out_ref.at[pl.ds(base + i * _CH, _CH), :], wsem[b])

        g = [gather(i, i) for i in range(_NBUF)]
        w = [None] * _NBUF
        for i in range(_NCHUNK):
            b = i % _NBUF
            if i > 0:
                bp = (i - 1) % _NBUF
                nxt = (i - 1) + _NBUF
                if nxt < _NCHUNK:
                    w[bp].wait()
                    w[bp] = None
                    g[bp] = gather(nxt, bp)
            g[b].wait()
            w[b] = write(i, b)
        for b in range(_NBUF):
            if w[b] is not None:
                w[b].wait()

        @pl.when(wid == 0)
        def _drain_mask():
            for c in mask_copy:
                c.wait()

    return k(x_flat, flat_idx)


def kernel(x):
    x_flat = x.reshape(_B * _T, _D)
    out_flat, mask_i = _sc_gather(x_flat, jnp.asarray(_FLAT_IDX))
    return out_flat.reshape(_B, _N_KEEP, _D), mask_i.astype(bool)


# speedup vs baseline: 1.0095x; 1.0095x over previous
import functools

import numpy as np
import jax
import jax.numpy as jnp
from jax import lax
from jax.experimental import pallas as pl
from jax.experimental.pallas import tpu as pltpu
from jax.experimental.pallas import tpu_sc as plsc

_MASK_RATIO = 0.75
_B, _T, _D = 64, 1024, 768
_N_MASK = int(_T * _MASK_RATIO)
_N_KEEP = _T - _N_MASK

_ROWS = _B * _N_KEEP
_NC, _NS = 2, 16
_NW = _NC * _NS
_RPW = _ROWS // _NW
_CH = 16
_NBUF = 8
_NLAG = 4
_NCHUNK = _RPW // _CH


_KEEP_IDX = np.asarray([
    2, 4, 5, 7, 16, 19, 29, 31, 34, 35, 37, 44, 45, 58, 61, 63, 65, 72,
    78, 82, 83, 85, 90, 99, 101, 102, 108, 110, 111, 112, 114, 117, 121,
    123, 129, 130, 139, 142, 144, 148, 152, 155, 156, 157, 163, 167, 174,
    175, 176, 177, 178, 179, 183, 188, 189, 197, 211, 212, 240, 251, 254,
    257, 259, 263, 268, 269, 272, 277, 278, 284, 291, 300, 302, 304, 305,
    309, 312, 315, 318, 323, 325, 336, 339, 350, 356, 363, 366, 367, 369,
    379, 388, 398, 409, 410, 415, 417, 429, 436, 441, 444, 446, 447, 448,
    452, 461, 462, 463, 480, 481, 487, 493, 495, 499, 501, 504, 507, 509,
    514, 516, 517, 518, 520, 524, 525, 532, 538, 540, 541, 542, 543, 544,
    551, 552, 553, 557, 562, 564, 565, 567, 569, 575, 577, 578, 580, 582,
    584, 585, 589, 590, 591, 598, 600, 602, 603, 605, 607, 617, 619, 638,
    649, 650, 654, 659, 670, 673, 675, 681, 690, 693, 694, 698, 703, 704,
    706, 707, 708, 709, 712, 714, 715, 730, 736, 739, 748, 750, 752, 753,
    755, 762, 765, 768, 769, 771, 774, 776, 777, 780, 787, 790, 792, 793,
    799, 803, 804, 808, 810, 816, 829, 836, 842, 846, 848, 854, 857, 859,
    864, 872, 874, 879, 883, 885, 893, 895, 901, 904, 910, 911, 914, 918,
    921, 928, 932, 934, 940, 942, 955, 957, 962, 966, 970, 973, 976, 981,
    984, 995, 996, 999, 1001, 1005, 1009, 1010, 1012, 1016, 1017, 1020,
    1021,
], dtype=np.int32)
_FLAT_IDX = (np.arange(_B, dtype=np.int64)[:, None] * _T
             + _KEEP_IDX[None, :].astype(np.int64)).reshape(-1).astype(np.int32)

_MASK_NP = np.ones(_T, dtype=bool)
_MASK_NP[_KEEP_IDX] = False
_MASK_BITS = [int(sum(int(_MASK_NP[16 * j + l]) << l for l in range(16)))
              for j in range(_T // 16)]


def _sc_gather(x_flat, flat_idx):
    mesh = plsc.VectorSubcoreMesh(core_axis_name="c", subcore_axis_name="s")

    @functools.partial(
        pl.kernel,
        out_type=(
            jax.ShapeDtypeStruct((_ROWS, _D), jnp.float32),
            jax.ShapeDtypeStruct((_T,), jnp.int32),
        ),
        mesh=mesh,
        scratch_types=(
            [pltpu.VMEM((_RPW,), jnp.int32)]
            + [pltpu.VMEM((_CH, _D), jnp.float32) for _ in range(_NBUF)]
            + [pltpu.VMEM((_T,), jnp.int32)]
            + [pltpu.SemaphoreType.DMA for _ in range(2 * _NBUF + 1)]
        ),
    )
    def k(x_ref, idx_ref, out_ref, mask_ref, idx_v, *rest):
        buf = list(rest[:_NBUF])
        m_v = rest[_NBUF]
        gsem = list(rest[_NBUF + 1:2 * _NBUF + 1])
        wsem = list(rest[2 * _NBUF + 1:3 * _NBUF + 1])
        msem = rest[3 * _NBUF + 1]
        wid = lax.axis_index("c") * _NS + lax.axis_index("s")
        base = wid * _RPW
        pltpu.sync_copy(idx_ref.at[pl.ds(base, _RPW)], idx_v)

        mask_copy = []

        @pl.when(wid == 0)
        def _build_mask():
            lane = lax.iota(jnp.int32, 16)
            for j in range(_T // 16):
                bits = jnp.full((16,), _MASK_BITS[j], jnp.int32)
                m_v[pl.ds(j * 16, 16)] = (bits >> lane) & 1
            mask_copy.append(pltpu.async_copy(m_v, mask_ref, msem))

        def gather(i, b):
            return pltpu.async_copy(
                x_ref.at[idx_v.at[pl.ds(i * _CH, _CH)]], buf[b], gsem[b])

        def write(i, b):
            return pltpu.async_copy(
                buf[b], out_ref.at[pl.ds(base + i * _CH, _CH), :], wsem[b])

        g = [gather(i, i) for i in range(_NBUF)]
        w = [None] * _NBUF
        for i in range(_NCHUNK):
            b = i % _NBUF
            if i >= _NLAG:
                j = i - _NLAG
                bj = j % _NBUF
                nxt = j + _NBUF
                if nxt < _NCHUNK:
                    w[bj].wait()
                    w[bj] = None
                    g[bj] = gather(nxt, bj)
            g[b].wait()
            w[b] = write(i, b)
        for b in range(_NBUF):
            if w[b] is not None:
                w[b].wait()

        @pl.when(wid == 0)
        def _drain_mask():
            for c in mask_copy:
                c.wait()

    return k(x_flat, flat_idx)


def kernel(x):
    x_flat = x.reshape(_B * _T, _D)
    out_flat, mask_i = _sc_gather(x_flat, jnp.asarray(_FLAT_IDX))
    return out_flat.reshape(_B, _N_KEEP, _D), mask_i.astype(bool)
